# baseline (speedup 1.0000x reference)
import jax
import jax.numpy as jnp
from jax.experimental import pallas as pl
from jax.experimental.pallas import tpu as pltpu


def _vq_kernel(x_ref, cb_ref, q_ref, idx_ref, loss_ref):
    C, H, W = x_ref.shape[1:]
    xb = x_ref[0].reshape(C, H * W)
    cb = cb_ref[...]
    K = cb.shape[0]
    HW = xb.shape[1]

    ys = jnp.sum(cb * cb, axis=1, keepdims=True)
    dots = jax.lax.dot_general(
        cb, xb, (((1,), (0,)), ((), ())),
        preferred_element_type=jnp.float32)
    dist = ys - 2.0 * dots

    mind = jnp.min(dist, axis=0, keepdims=True)
    iota = jax.lax.broadcasted_iota(jnp.int32, (K, HW), 0)
    idx = jnp.min(jnp.where(dist == mind, iota, K), axis=0)
    idx_ref[0, 0, :] = idx

    onehot = (iota == idx[None, :]).astype(jnp.bfloat16)
    q = jax.lax.dot_general(
        cb.astype(jnp.bfloat16), onehot, (((0,), (0,)), ((), ())),
        preferred_element_type=jnp.float32)
    q_ref[0] = q.reshape(C, H, W)

    diff = xb - q
    loss_ref[...] = jnp.sum(diff * diff).reshape(1, 1, 1)


def kernel(x, codebook):
    B, C, H, W = x.shape
    K = codebook.shape[0]
    HW = H * W

    quantized, idx, loss_parts = pl.pallas_call(
        _vq_kernel,
        grid=(B,),
        in_specs=[
            pl.BlockSpec((1, C, H, W), lambda b: (b, 0, 0, 0)),
            pl.BlockSpec((K, C), lambda b: (0, 0)),
        ],
        out_specs=[
            pl.BlockSpec((1, C, H, W), lambda b: (b, 0, 0, 0)),
            pl.BlockSpec((1, 1, HW), lambda b: (b, 0, 0)),
            pl.BlockSpec((1, 1, 1), lambda b: (b, 0, 0)),
        ],
        out_shape=[
            jax.ShapeDtypeStruct((B, C, H, W), jnp.float32),
            jax.ShapeDtypeStruct((B, 1, HW), jnp.int32),
            jax.ShapeDtypeStruct((B, 1, 1), jnp.float32),
        ],
        compiler_params=pltpu.CompilerParams(
            dimension_semantics=("parallel",),
        ),
    )(x, codebook)
    indexes = idx.reshape(B * H * W)
    loss = jnp.sum(loss_parts) / (B * C * H * W)
    return quantized, indexes, loss


# speedup vs baseline: 2.3089x; 2.3089x over previous
import jax
import jax.numpy as jnp
from jax.experimental import pallas as pl
from jax.experimental.pallas import tpu as pltpu


def _vq_kernel(x_ref, cb_ref, q_ref, idx_ref, loss_ref, h_ref, cbb_ref):
    b = pl.program_id(0)
    xb = x_ref[0]
    cb = cb_ref[...]
    K = cb.shape[0]
    HW = xb.shape[1]

    @pl.when(b == 0)
    def _precompute():
        h_ref[...] = 0.5 * jnp.sum(cb * cb, axis=1, keepdims=True)
        cbb_ref[...] = cb.astype(jnp.bfloat16)

    dots = jax.lax.dot_general(
        cb, xb, (((1,), (0,)), ((), ())),
        preferred_element_type=jnp.float32)
    t = h_ref[...] - dots

    mind = jnp.min(t, axis=0, keepdims=True)
    iota = jax.lax.broadcasted_iota(jnp.int32, (K, HW), 0)
    idx = jnp.min(jnp.where(t == mind, iota, K), axis=0)
    idx_ref[0, 0, :] = idx

    onehot = (iota == idx[None, :]).astype(jnp.bfloat16)
    q = jax.lax.dot_general(
        cbb_ref[...], onehot, (((0,), (0,)), ((), ())),
        preferred_element_type=jnp.float32)
    q_ref[0] = q

    xs = jnp.sum(xb * xb, axis=0, keepdims=True)
    part = jnp.sum(xs + 2.0 * mind).reshape(1, 1)

    @pl.when(b == 0)
    def _init():
        loss_ref[...] = jnp.zeros_like(loss_ref)

    loss_ref[...] += part


def kernel(x, codebook):
    B, C, H, W = x.shape
    K = codebook.shape[0]
    HW = H * W
    xr = x.reshape(B, C, HW)

    q, idx, loss_sum = pl.pallas_call(
        _vq_kernel,
        grid=(B,),
        in_specs=[
            pl.BlockSpec((1, C, HW), lambda b: (b, 0, 0)),
            pl.BlockSpec((K, C), lambda b: (0, 0)),
        ],
        out_specs=[
            pl.BlockSpec((1, C, HW), lambda b: (b, 0, 0)),
            pl.BlockSpec((1, 1, HW), lambda b: (b, 0, 0)),
            pl.BlockSpec((1, 1), lambda b: (0, 0)),
        ],
        out_shape=[
            jax.ShapeDtypeStruct((B, C, HW), jnp.float32),
            jax.ShapeDtypeStruct((B, 1, HW), jnp.int32),
            jax.ShapeDtypeStruct((1, 1), jnp.float32),
        ],
        scratch_shapes=[
            pltpu.VMEM((K, 1), jnp.float32),
            pltpu.VMEM((K, C), jnp.bfloat16),
        ],
    )(xr, codebook)

    quantized = q.reshape(B, C, H, W)
    indexes = idx.reshape(B * H * W)
    loss = loss_sum[0, 0] / (B * C * H * W)
    return quantized, indexes, loss


# speedup vs baseline: 2.4688x; 1.0692x over previous
import jax
import jax.numpy as jnp
from jax.experimental import pallas as pl
from jax.experimental.pallas import tpu as pltpu


def _vq_kernel(x_ref, cb_ref, q_ref, idx_ref, loss_ref, h_ref, cbb_ref):
    b = pl.program_id(0)
    xb = x_ref[0]
    cb = cb_ref[...]
    K = cb.shape[0]
    HW = xb.shape[1]

    @pl.when(b == 0)
    def _precompute():
        h_ref[...] = 0.5 * jnp.sum(cb * cb, axis=1, keepdims=True)
        cbb_ref[...] = cb.astype(jnp.bfloat16)

    dots = jax.lax.dot_general(
        cb, xb, (((1,), (0,)), ((), ())),
        preferred_element_type=jnp.float32)
    t = h_ref[...] - dots

    idx = jnp.argmin(t, axis=0)
    idx_ref[0, 0, :] = idx

    iota = jax.lax.broadcasted_iota(jnp.int32, (K, HW), 0)
    onehot = (iota == idx[None, :]).astype(jnp.bfloat16)
    q = jax.lax.dot_general(
        cbb_ref[...], onehot, (((0,), (0,)), ((), ())),
        preferred_element_type=jnp.float32)
    q_ref[0] = q

    diff = xb - q
    part = jnp.sum(diff * diff).reshape(1, 1)

    @pl.when(b == 0)
    def _init():
        loss_ref[...] = jnp.zeros_like(loss_ref)

    loss_ref[...] += part


def kernel(x, codebook):
    B, C, H, W = x.shape
    K = codebook.shape[0]
    HW = H * W
    xr = x.reshape(B, C, HW)

    q, idx, loss_sum = pl.pallas_call(
        _vq_kernel,
        grid=(B,),
        in_specs=[
            pl.BlockSpec((1, C, HW), lambda b: (b, 0, 0)),
            pl.BlockSpec((K, C), lambda b: (0, 0)),
        ],
        out_specs=[
            pl.BlockSpec((1, C, HW), lambda b: (b, 0, 0)),
            pl.BlockSpec((1, 1, HW), lambda b: (b, 0, 0)),
            pl.BlockSpec((1, 1), lambda b: (0, 0)),
        ],
        out_shape=[
            jax.ShapeDtypeStruct((B, C, HW), jnp.float32),
            jax.ShapeDtypeStruct((B, 1, HW), jnp.int32),
            jax.ShapeDtypeStruct((1, 1), jnp.float32),
        ],
        scratch_shapes=[
            pltpu.VMEM((K, 1), jnp.float32),
            pltpu.VMEM((K, C), jnp.bfloat16),
        ],
    )(xr, codebook)

    quantized = q.reshape(B, C, H, W)
    indexes = idx.reshape(B * H * W)
    loss = loss_sum[0, 0] / (B * C * H * W)
    return quantized, indexes, loss


# speedup vs baseline: 2.5085x; 1.0161x over previous
import jax
import jax.numpy as jnp
from jax.experimental import pallas as pl
from jax.experimental.pallas import tpu as pltpu


def _vq_kernel(x_ref, cb_ref, q_ref, idx_ref, loss_ref, h_ref, cbb_ref):
    b = pl.program_id(0)
    xb = x_ref[0]
    cb = cb_ref[...]
    K = cb.shape[0]
    HW = xb.shape[1]

    @pl.when(b == 0)
    def _precompute():
        h_ref[...] = 0.5 * jnp.sum(cb * cb, axis=1, keepdims=True)
        cbb_ref[...] = cb.astype(jnp.bfloat16)

    dots = jax.lax.dot_general(
        cb, xb, (((1,), (0,)), ((), ())),
        preferred_element_type=jnp.float32)
    idx = jnp.argmin(h_ref[...] - dots, axis=0)
    idx_ref[0, 0, :] = idx

    iota = jax.lax.broadcasted_iota(jnp.int32, (K, HW), 0)
    onehot = (iota == idx[None, :]).astype(jnp.bfloat16)
    q = jax.lax.dot_general(
        cbb_ref[...], onehot, (((0,), (0,)), ((), ())),
        preferred_element_type=jnp.float32)
    q_ref[0] = q.astype(jnp.bfloat16)

    diff = xb - q
    part = jnp.sum(diff * diff).reshape(1, 1)

    @pl.when(b == 0)
    def _init():
        loss_ref[...] = jnp.zeros_like(loss_ref)

    loss_ref[...] += part


def kernel(x, codebook):
    B, C, H, W = x.shape
    K = codebook.shape[0]
    HW = H * W
    xr = x.reshape(B, C, HW)

    q, idx, loss_sum = pl.pallas_call(
        _vq_kernel,
        grid=(B,),
        in_specs=[
            pl.BlockSpec((1, C, HW), lambda b: (b, 0, 0)),
            pl.BlockSpec((K, C), lambda b: (0, 0)),
        ],
        out_specs=[
            pl.BlockSpec((1, C, HW), lambda b: (b, 0, 0)),
            pl.BlockSpec((1, 1, HW), lambda b: (b, 0, 0)),
            pl.BlockSpec((1, 1), lambda b: (0, 0)),
        ],
        out_shape=[
            jax.ShapeDtypeStruct((B, C, HW), jnp.bfloat16),
            jax.ShapeDtypeStruct((B, 1, HW), jnp.int32),
            jax.ShapeDtypeStruct((1, 1), jnp.float32),
        ],
        scratch_shapes=[
            pltpu.VMEM((K, 1), jnp.float32),
            pltpu.VMEM((K, C), jnp.bfloat16),
        ],
    )(xr, codebook)

    quantized = q.astype(jnp.float32).reshape(B, C, H, W)
    indexes = idx.reshape(B * H * W)
    loss = loss_sum[0, 0] / (B * C * H * W)
    return quantized, indexes, loss


# speedup vs baseline: 2.7948x; 1.1142x over previous
import jax
import jax.numpy as jnp
from jax.experimental import pallas as pl
from jax.experimental.pallas import tpu as pltpu


def _vq_kernel(x_ref, cb_ref, q_ref, idx_ref, loss_ref, h_ref, cbb_ref):
    b = pl.program_id(0)
    cb = cb_ref[...]
    K = cb.shape[0]
    HW = x_ref.shape[2]
    NB = x_ref.shape[0]

    @pl.when(b == 0)
    def _precompute():
        h_ref[...] = 0.5 * jnp.sum(cb * cb, axis=1, keepdims=True)
        cbb_ref[...] = cb.astype(jnp.bfloat16)

    @pl.when(b == 0)
    def _init():
        loss_ref[...] = jnp.zeros_like(loss_ref)

    for i in range(NB):
        xb = x_ref[i]
        dots = jax.lax.dot_general(
            cb, xb, (((1,), (0,)), ((), ())),
            preferred_element_type=jnp.float32)
        idx = jnp.argmin(h_ref[...] - dots, axis=0)
        idx_ref[i, 0, :] = idx

        iota = jax.lax.broadcasted_iota(jnp.int32, (K, HW), 0)
        onehot = (iota == idx[None, :]).astype(jnp.bfloat16)
        q = jax.lax.dot_general(
            cbb_ref[...], onehot, (((0,), (0,)), ((), ())),
            preferred_element_type=jnp.float32)
        q_ref[i] = q.astype(jnp.bfloat16)

        diff = xb - q
        loss_ref[...] += jnp.sum(diff * diff).reshape(1, 1)


def kernel(x, codebook):
    B, C, H, W = x.shape
    K = codebook.shape[0]
    HW = H * W
    xr = x.reshape(B, C, HW)

    NB = 2
    q, idx, loss_sum = pl.pallas_call(
        _vq_kernel,
        grid=(B // NB,),
        in_specs=[
            pl.BlockSpec((NB, C, HW), lambda b: (b, 0, 0)),
            pl.BlockSpec((K, C), lambda b: (0, 0)),
        ],
        out_specs=[
            pl.BlockSpec((NB, C, HW), lambda b: (b, 0, 0)),
            pl.BlockSpec((NB, 1, HW), lambda b: (b, 0, 0)),
            pl.BlockSpec((1, 1), lambda b: (0, 0)),
        ],
        out_shape=[
            jax.ShapeDtypeStruct((B, C, HW), jnp.bfloat16),
            jax.ShapeDtypeStruct((B, 1, HW), jnp.int32),
            jax.ShapeDtypeStruct((1, 1), jnp.float32),
        ],
        scratch_shapes=[
            pltpu.VMEM((K, 1), jnp.float32),
            pltpu.VMEM((K, C), jnp.bfloat16),
        ],
    )(xr, codebook)

    quantized = q.astype(jnp.float32).reshape(B, C, H, W)
    indexes = idx.reshape(B * H * W)
    loss = loss_sum[0, 0] / (B * C * H * W)
    return quantized, indexes, loss


# speedup vs baseline: 2.9278x; 1.0476x over previous
import jax
import jax.numpy as jnp
from jax.experimental import pallas as pl
from jax.experimental.pallas import tpu as pltpu


def _vq_kernel(x_ref, cb_ref, q_ref, idx_ref, loss_ref, h_ref, cbb_ref):
    b = pl.program_id(0)
    cb = cb_ref[...]
    K = cb.shape[0]
    HW = x_ref.shape[2]
    NB = x_ref.shape[0]

    @pl.when(b == 0)
    def _precompute():
        h_ref[...] = 0.5 * jnp.sum(cb * cb, axis=1, keepdims=True)
        cbb_ref[...] = cb.astype(jnp.bfloat16)

    @pl.when(b == 0)
    def _init():
        loss_ref[...] = jnp.zeros_like(loss_ref)

    for i in range(NB):
        xb = x_ref[i]
        dots = jax.lax.dot_general(
            cb, xb, (((1,), (0,)), ((), ())),
            preferred_element_type=jnp.float32)
        idx = jnp.argmin(h_ref[...] - dots, axis=0)
        idx_ref[i, 0, :] = idx

        iota = jax.lax.broadcasted_iota(jnp.int32, (K, HW), 0)
        onehot = (iota == idx[None, :]).astype(jnp.bfloat16)
        q = jax.lax.dot_general(
            cbb_ref[...], onehot, (((0,), (0,)), ((), ())),
            preferred_element_type=jnp.float32)
        q_ref[i] = q.astype(jnp.bfloat16)

        diff = xb - q
        loss_ref[...] += jnp.sum(diff * diff).reshape(1, 1)


def kernel(x, codebook):
    B, C, H, W = x.shape
    K = codebook.shape[0]
    HW = H * W
    xr = x.reshape(B, C, HW)

    NB = 4
    q, idx, loss_sum = pl.pallas_call(
        _vq_kernel,
        grid=(B // NB,),
        in_specs=[
            pl.BlockSpec((NB, C, HW), lambda b: (b, 0, 0)),
            pl.BlockSpec((K, C), lambda b: (0, 0)),
        ],
        out_specs=[
            pl.BlockSpec((NB, C, HW), lambda b: (b, 0, 0)),
            pl.BlockSpec((NB, 1, HW), lambda b: (b, 0, 0)),
            pl.BlockSpec((1, 1), lambda b: (0, 0)),
        ],
        out_shape=[
            jax.ShapeDtypeStruct((B, C, HW), jnp.bfloat16),
            jax.ShapeDtypeStruct((B, 1, HW), jnp.int32),
            jax.ShapeDtypeStruct((1, 1), jnp.float32),
        ],
        scratch_shapes=[
            pltpu.VMEM((K, 1), jnp.float32),
            pltpu.VMEM((K, C), jnp.bfloat16),
        ],
    )(xr, codebook)

    quantized = q.astype(jnp.float32).reshape(B, C, H, W)
    indexes = idx.reshape(B * H * W)
    loss = loss_sum[0, 0] / (B * C * H * W)
    return quantized, indexes, loss


# speedup vs baseline: 2.9360x; 1.0028x over previous
import jax
import jax.numpy as jnp
from jax.experimental import pallas as pl
from jax.experimental.pallas import tpu as pltpu


def _vq_kernel(x_ref, cb_ref, q_ref, idx_ref, loss_ref, h_ref, cbb_ref):
    b = pl.program_id(0)
    cb = cb_ref[...]
    K = cb.shape[0]
    HW = x_ref.shape[2]
    NB = x_ref.shape[0]

    @pl.when(b == 0)
    def _precompute():
        h_ref[...] = 0.5 * jnp.sum(cb * cb, axis=1, keepdims=True)
        cbb_ref[...] = cb.astype(jnp.bfloat16)

    @pl.when(b == 0)
    def _init():
        loss_ref[...] = jnp.zeros_like(loss_ref)

    for i in range(NB):
        xb = x_ref[i]
        dots = jax.lax.dot_general(
            cb, xb, (((1,), (0,)), ((), ())),
            preferred_element_type=jnp.float32)
        idx = jnp.argmin(h_ref[...] - dots, axis=0)
        idx_ref[i, 0, :] = idx

        iota = jax.lax.broadcasted_iota(jnp.int32, (K, HW), 0)
        onehot = (iota == idx[None, :]).astype(jnp.bfloat16)
        q = jax.lax.dot_general(
            cbb_ref[...], onehot, (((0,), (0,)), ((), ())),
            preferred_element_type=jnp.float32)
        q_ref[i] = q.astype(jnp.bfloat16)

        diff = xb - q
        loss_ref[...] += jnp.sum(diff * diff).reshape(1, 1)


def kernel(x, codebook):
    B, C, H, W = x.shape
    K = codebook.shape[0]
    HW = H * W
    xr = x.reshape(B, C, HW)

    NB = 8
    q, idx, loss_sum = pl.pallas_call(
        _vq_kernel,
        grid=(B // NB,),
        in_specs=[
            pl.BlockSpec((NB, C, HW), lambda b: (b, 0, 0)),
            pl.BlockSpec((K, C), lambda b: (0, 0)),
        ],
        out_specs=[
            pl.BlockSpec((NB, C, HW), lambda b: (b, 0, 0)),
            pl.BlockSpec((NB, 1, HW), lambda b: (b, 0, 0)),
            pl.BlockSpec((1, 1), lambda b: (0, 0)),
        ],
        out_shape=[
            jax.ShapeDtypeStruct((B, C, HW), jnp.bfloat16),
            jax.ShapeDtypeStruct((B, 1, HW), jnp.int32),
            jax.ShapeDtypeStruct((1, 1), jnp.float32),
        ],
        scratch_shapes=[
            pltpu.VMEM((K, 1), jnp.float32),
            pltpu.VMEM((K, C), jnp.bfloat16),
        ],
    )(xr, codebook)

    quantized = q.astype(jnp.float32).reshape(B, C, H, W)
    indexes = idx.reshape(B * H * W)
    loss = loss_sum[0, 0] / (B * C * H * W)
    return quantized, indexes, loss
